# baseline (speedup 1.0000x reference)
import jax
import jax.numpy as jnp
from jax.experimental import pallas as pl
from jax.experimental.pallas import tpu as pltpu

_BM = 2000


def _heads_kernel(x_ref, w_ref, b_ref, cls_ref, box_ref, obj_ref):
    acc = jnp.dot(x_ref[...], w_ref[...], preferred_element_type=jnp.float32)
    acc = acc + b_ref[...]
    n_cls = cls_ref.shape[1]
    n_box = box_ref.shape[1]
    cls_ref[...] = acc[:, :n_cls]
    box_ref[...] = acc[:, n_cls:n_cls + n_box]
    obj_ref[...] = acc[:, n_cls + n_box:n_cls + n_box + obj_ref.shape[1]]


def kernel(x, W_cls, b_cls, W_box, b_box, W_obj, b_obj):
    M, K = x.shape
    n_cls = W_cls.shape[1]
    n_box = W_box.shape[1]
    n_obj = W_obj.shape[1]
    n_all = n_cls + n_box + n_obj

    W = jnp.concatenate([W_cls, W_box, W_obj], axis=1)
    b = jnp.concatenate([b_cls, b_box, b_obj])[None, :]

    bm = _BM if M % _BM == 0 else M
    grid = (M // bm,)

    cls_out, box_out, obj_out = pl.pallas_call(
        _heads_kernel,
        grid=grid,
        in_specs=[
            pl.BlockSpec((bm, K), lambda i: (i, 0)),
            pl.BlockSpec((K, n_all), lambda i: (0, 0)),
            pl.BlockSpec((1, n_all), lambda i: (0, 0)),
        ],
        out_specs=[
            pl.BlockSpec((bm, n_cls), lambda i: (i, 0)),
            pl.BlockSpec((bm, n_box), lambda i: (i, 0)),
            pl.BlockSpec((bm, n_obj), lambda i: (i, 0)),
        ],
        out_shape=[
            jax.ShapeDtypeStruct((M, n_cls), x.dtype),
            jax.ShapeDtypeStruct((M, n_box), x.dtype),
            jax.ShapeDtypeStruct((M, n_obj), x.dtype),
        ],
        compiler_params=pltpu.CompilerParams(
            dimension_semantics=("parallel",),
        ),
    )(x, W, b)
    return (cls_out, box_out, obj_out)


# speedup vs baseline: 1.0420x; 1.0420x over previous
import jax
import jax.numpy as jnp
from jax.experimental import pallas as pl
from jax.experimental.pallas import tpu as pltpu

_BM = 4000


def _heads_kernel(x_ref, w_ref, b_ref, cls_ref, box_ref, obj_ref):
    acc = jnp.dot(x_ref[...], w_ref[...], preferred_element_type=jnp.float32)
    acc = acc + b_ref[...]
    n_cls = cls_ref.shape[1]
    n_box = box_ref.shape[1]
    cls_ref[...] = acc[:, :n_cls]
    box_ref[...] = acc[:, n_cls:n_cls + n_box]
    obj_ref[...] = acc[:, n_cls + n_box:n_cls + n_box + obj_ref.shape[1]]


def kernel(x, W_cls, b_cls, W_box, b_box, W_obj, b_obj):
    M, K = x.shape
    n_cls = W_cls.shape[1]
    n_box = W_box.shape[1]
    n_obj = W_obj.shape[1]
    n_all = n_cls + n_box + n_obj

    W = jnp.concatenate([W_cls, W_box, W_obj], axis=1)
    b = jnp.concatenate([b_cls, b_box, b_obj])[None, :]

    bm = _BM if M % _BM == 0 else M
    grid = (M // bm,)

    cls_out, box_out, obj_out = pl.pallas_call(
        _heads_kernel,
        grid=grid,
        in_specs=[
            pl.BlockSpec((bm, K), lambda i: (i, 0)),
            pl.BlockSpec((K, n_all), lambda i: (0, 0)),
            pl.BlockSpec((1, n_all), lambda i: (0, 0)),
        ],
        out_specs=[
            pl.BlockSpec((bm, n_cls), lambda i: (i, 0)),
            pl.BlockSpec((bm, n_box), lambda i: (i, 0)),
            pl.BlockSpec((bm, n_obj), lambda i: (i, 0)),
        ],
        out_shape=[
            jax.ShapeDtypeStruct((M, n_cls), x.dtype),
            jax.ShapeDtypeStruct((M, n_box), x.dtype),
            jax.ShapeDtypeStruct((M, n_obj), x.dtype),
        ],
        compiler_params=pltpu.CompilerParams(
            dimension_semantics=("parallel",),
        ),
    )(x, W, b)
    return (cls_out, box_out, obj_out)
